# baseline (speedup 1.0000x reference)
import functools

import jax
import jax.numpy as jnp
from jax import lax
from jax.experimental import pallas as pl
from jax.experimental.pallas import tpu as pltpu
from jax.experimental.pallas import tpu_sc as plsc

N_NODES = 10000
D = 128
E = 320000

NC = 2
NS = 16
NW = NC * NS

CHUNK = 128
CPT = 80
EPT = CPT * CHUNK
E_PAD = NW * EPT
ROWS_PER_TILE = 632
ACC_ROWS = NS * ROWS_PER_TILE
DEG_RPT = 640
DEG_ROWS = NS * DEG_RPT
TRASH = N_NODES

_mesh = plsc.VectorSubcoreMesh(core_axis_name="c", subcore_axis_name="s")


@functools.partial(
    pl.kernel,
    out_type=jax.ShapeDtypeStruct((NC, DEG_ROWS), jnp.float32),
    mesh=_mesh,
    scratch_types=[
        pltpu.VMEM((CHUNK,), jnp.float32),
        pltpu.VMEM((CPT, CHUNK), jnp.int32),
        pltpu.VMEM_SHARED((DEG_ROWS,), jnp.float32),
    ],
)
def _deg_sc(dst_hbm, zeros_hbm, out_hbm, ones_v, didx, acc):
    cid = lax.axis_index("c")
    sid = lax.axis_index("s")
    wid = sid * NC + cid
    for j in range(CHUNK // 16):
        ones_v[pl.ds(j * 16, 16)] = jnp.ones((16,), jnp.float32)
    pltpu.sync_copy(zeros_hbm, acc.at[pl.ds(sid * DEG_RPT, DEG_RPT)])
    pltpu.sync_copy(dst_hbm.at[wid], didx)
    plsc.subcore_barrier()

    def body(c, carry):
        pltpu.sync_copy(ones_v, acc.at[didx.at[c]], add=True)
        return carry

    lax.fori_loop(0, CPT, body, 0)
    plsc.subcore_barrier()
    pltpu.sync_copy(
        acc.at[pl.ds(sid * DEG_RPT, DEG_RPT)],
        out_hbm.at[cid, pl.ds(sid * DEG_RPT, DEG_RPT)],
    )


@functools.partial(
    pl.kernel,
    out_type=jax.ShapeDtypeStruct((NC, ACC_ROWS, D), jnp.float32),
    mesh=_mesh,
    scratch_types=[
        pltpu.VMEM((8, CHUNK), jnp.int32),
        pltpu.VMEM((8, CHUNK), jnp.int32),
        pltpu.VMEM((CPT, CHUNK), jnp.int32),
        pltpu.VMEM((CHUNK, D), jnp.float32),
        pltpu.VMEM((CHUNK, D), jnp.float32),
        pltpu.SemaphoreType.DMA,
        pltpu.SemaphoreType.DMA,
        pltpu.SemaphoreType.DMA,
        pltpu.SemaphoreType.DMA,
        pltpu.VMEM_SHARED((ACC_ROWS, D), jnp.float32),
    ],
)
def _agg_sc(y_hbm, src_hbm, dst_hbm, zeros_hbm, out_hbm,
            st_a, st_b, didx, rows_a, rows_b,
            sem_ra, sem_rb, sem_ia, sem_ib, acc):
    cid = lax.axis_index("c")
    sid = lax.axis_index("s")
    wid = sid * NC + cid
    pltpu.sync_copy(zeros_hbm, acc.at[pl.ds(sid * ROWS_PER_TILE, ROWS_PER_TILE)])
    pltpu.sync_copy(dst_hbm.at[wid], didx)

    st = (st_a, st_b)
    rows = (rows_a, rows_b)
    sem_r = (sem_ra, sem_rb)
    sem_i = (sem_ia, sem_ib)

    plsc.subcore_barrier()

    pltpu.sync_copy(src_hbm.at[wid, 0], st_a.at[pl.ds(0, 1)])
    pltpu.async_copy(y_hbm.at[st_a.at[0]], rows_a, sem_ra)
    pltpu.async_copy(src_hbm.at[wid, 1], st_b.at[pl.ds(0, 1)], sem_ib)

    def body(c, carry):
        k = lax.rem(c, 2)

        def chunk(k):
            @pl.when(c + 1 < CPT)
            def _():
                pltpu.make_async_copy(
                    src_hbm.at[wid, 0], st[1 - k].at[pl.ds(0, 1)],
                    sem_i[1 - k]).wait()
                pltpu.async_copy(y_hbm.at[st[1 - k].at[0]], rows[1 - k],
                                 sem_r[1 - k])
            pltpu.make_async_copy(y_hbm.at[st[k].at[0]], rows[k],
                                  sem_r[k]).wait()

            @pl.when(c + 2 < CPT)
            def _():
                pltpu.async_copy(src_hbm.at[wid, c + 2],
                                 st[k].at[pl.ds(0, 1)], sem_i[k])
            pltpu.sync_copy(rows[k], acc.at[didx.at[c]], add=True)

        @pl.when(k == 0)
        def _():
            chunk(0)

        @pl.when(k == 1)
        def _():
            chunk(1)

        return carry

    lax.fori_loop(0, CPT, body, 0)
    plsc.subcore_barrier()
    pltpu.sync_copy(
        acc.at[pl.ds(sid * ROWS_PER_TILE, ROWS_PER_TILE)],
        out_hbm.at[cid, pl.ds(sid * ROWS_PER_TILE, ROWS_PER_TILE)],
    )


def kernel(x, edge_index, W1, b1, W2, b2):
    src = edge_index[0].astype(jnp.int32)
    dst = edge_index[1].astype(jnp.int32)
    pad = E_PAD - E
    srcp = jnp.concatenate([src, jnp.zeros((pad,), jnp.int32)])
    dstp = jnp.concatenate([dst, jnp.full((pad,), TRASH, jnp.int32)])
    srcp = srcp.reshape(NW, CPT, 1, CHUNK)
    dstp = dstp.reshape(NW, CPT, CHUNK)

    zeros1 = jnp.zeros((DEG_RPT,), jnp.float32)
    zeros2 = jnp.zeros((ROWS_PER_TILE, D), jnp.float32)

    deg_parts = _deg_sc(dstp, zeros1)
    deg = deg_parts[0, :N_NODES] + deg_parts[1, :N_NODES] + 1.0
    dis = lax.rsqrt(deg)[:, None]

    y1 = (x @ W1) * dis
    agg1 = _agg_sc(y1, srcp, dstp, zeros2)
    h = dis * (agg1[0, :N_NODES] + agg1[1, :N_NODES] + y1) + b1
    h = jnp.maximum(h, 0.0)

    y2 = (h @ W2) * dis
    agg2 = _agg_sc(y2, srcp, dstp, zeros2)
    return dis * (agg2[0, :N_NODES] + agg2[1, :N_NODES] + y2) + b2
